# baseline (speedup 1.0000x reference)
import functools

import jax
import jax.numpy as jnp
from jax import lax
from jax.experimental import pallas as pl
from jax.experimental.pallas import tpu as pltpu
from jax.experimental.pallas import tpu_sc as plsc

VOCAB = 1000000
D_MODEL = 64
B_TOTAL = 16384 * 50
NUM_WORKERS = 32
B_PER_W = B_TOTAL // NUM_WORKERS
CHUNK = 640
NUM_CHUNKS = B_PER_W // CHUNK
NUM_GROUPS = NUM_CHUNKS // 2

VBLK = 32768
N_VBLK = -(-VOCAB // VBLK)
VOCAB_PAD = N_VBLK * VBLK

_mesh = plsc.VectorSubcoreMesh(core_axis_name="c", subcore_axis_name="s")


def _transpose_body(wt_ref, out_ref):
    a = wt_ref[:, : VBLK // 2]
    b = wt_ref[:, VBLK // 2 :]
    out_ref[:, :D_MODEL] = jnp.transpose(a)
    out_ref[:, D_MODEL:] = jnp.transpose(b)


_transpose_kernel = pl.pallas_call(
    _transpose_body,
    grid=(N_VBLK,),
    in_specs=[pl.BlockSpec((D_MODEL, VBLK), lambda i: (0, i))],
    out_specs=pl.BlockSpec((VBLK // 2, 2 * D_MODEL), lambda i: (i, 0)),
    out_shape=jax.ShapeDtypeStruct((VOCAB_PAD // 2, 2 * D_MODEL), jnp.float32),
    compiler_params=pltpu.CompilerParams(
        dimension_semantics=("arbitrary",),
    ),
)


@functools.partial(
    pl.kernel,
    mesh=_mesh,
    out_type=jax.ShapeDtypeStruct((B_TOTAL, D_MODEL), jnp.float32),
    scratch_types=[
        pltpu.VMEM((NUM_CHUNKS, CHUNK), jnp.int32),
        pltpu.VMEM((2, CHUNK, D_MODEL), jnp.float32),
        pltpu.SemaphoreType.DMA((2,)),
        pltpu.SemaphoreType.DMA((2,)),
    ],
    compiler_params=pltpu.CompilerParams(use_tc_tiling_on_sc=False),
)
def _gather_kernel(idx_hbm, table_hbm, out_hbm, idx_v, rows_v, gsem, wsem):
    wid = lax.axis_index("s") * 2 + lax.axis_index("c")
    base = wid * B_PER_W
    pltpu.sync_copy(idx_hbm.at[wid], idx_v)

    def gather_cp(j, b):
        return pltpu.make_async_copy(
            table_hbm.at[idx_v.at[j]], rows_v.at[b], gsem.at[b]
        )

    def write_cp(j, b):
        return pltpu.make_async_copy(
            rows_v.at[b], out_hbm.at[pl.ds(base + j * CHUNK, CHUNK)], wsem.at[b]
        )

    gather_cp(0, 0).start()
    gather_cp(1, 1).start()

    def body(g, carry):
        j0 = g * 2
        gather_cp(j0, 0).wait()
        write_cp(j0, 0).start()
        gather_cp(j0 + 1, 1).wait()
        write_cp(j0 + 1, 1).start()
        write_cp(j0, 0).wait()
        gather_cp(j0 + 2, 0).start()
        write_cp(j0 + 1, 1).wait()
        gather_cp(j0 + 3, 1).start()
        return carry

    lax.fori_loop(0, NUM_GROUPS - 1, body, 0)
    j0 = (NUM_GROUPS - 1) * 2
    gather_cp(j0, 0).wait()
    write_cp(j0, 0).start()
    gather_cp(j0 + 1, 1).wait()
    write_cp(j0 + 1, 1).start()
    write_cp(j0, 0).wait()
    write_cp(j0 + 1, 1).wait()


def kernel(x, weight):
    w2 = _transpose_kernel(weight.T)
    table = w2.reshape(VOCAB_PAD, D_MODEL)
    v = x.reshape(NUM_WORKERS, NUM_CHUNKS, CHUNK).astype(jnp.int32)
    q = v & (VBLK - 1)
    r = (v - q) + 2 * (q & (VBLK // 2 - 1)) + (q >> (VBLK // 2).bit_length() - 1)
    out = _gather_kernel(r, table)
    return out.reshape(x.shape + (D_MODEL,))


# speedup vs baseline: 1.0011x; 1.0011x over previous
import functools

import jax
import jax.numpy as jnp
from jax import lax
from jax.experimental import pallas as pl
from jax.experimental.pallas import tpu as pltpu
from jax.experimental.pallas import tpu_sc as plsc

VOCAB = 1000000
D_MODEL = 64
B_TOTAL = 16384 * 50
NUM_WORKERS = 32
B_PER_W = B_TOTAL // NUM_WORKERS
CHUNK = 512
NUM_CHUNKS = B_PER_W // CHUNK
NUM_GROUPS = NUM_CHUNKS // 2

VBLK = 32768
N_VBLK = -(-VOCAB // VBLK)
VOCAB_PAD = N_VBLK * VBLK

_mesh = plsc.VectorSubcoreMesh(core_axis_name="c", subcore_axis_name="s")


def _transpose_body(wt_ref, out_ref):
    a = wt_ref[:, : VBLK // 2]
    b = wt_ref[:, VBLK // 2 :]
    out_ref[:, :D_MODEL] = jnp.transpose(a)
    out_ref[:, D_MODEL:] = jnp.transpose(b)


_transpose_kernel = pl.pallas_call(
    _transpose_body,
    grid=(N_VBLK,),
    in_specs=[pl.BlockSpec((D_MODEL, VBLK), lambda i: (0, i))],
    out_specs=pl.BlockSpec((VBLK // 2, 2 * D_MODEL), lambda i: (i, 0)),
    out_shape=jax.ShapeDtypeStruct((VOCAB_PAD // 2, 2 * D_MODEL), jnp.float32),
    compiler_params=pltpu.CompilerParams(
        dimension_semantics=("arbitrary",),
    ),
)


@functools.partial(
    pl.kernel,
    mesh=_mesh,
    out_type=jax.ShapeDtypeStruct((B_TOTAL, D_MODEL), jnp.float32),
    scratch_types=[
        pltpu.VMEM((NUM_CHUNKS, CHUNK), jnp.int32),
        pltpu.VMEM((2, CHUNK, D_MODEL), jnp.float32),
        pltpu.SemaphoreType.DMA((2,)),
        pltpu.SemaphoreType.DMA((2,)),
    ],
    compiler_params=pltpu.CompilerParams(use_tc_tiling_on_sc=False),
)
def _gather_kernel(idx_hbm, table_hbm, out_hbm, idx_v, rows_v, gsem, wsem):
    wid = lax.axis_index("s") * 2 + lax.axis_index("c")
    base = wid * B_PER_W
    pltpu.sync_copy(idx_hbm.at[wid], idx_v)

    def gather_cp(j, b):
        return pltpu.make_async_copy(
            table_hbm.at[idx_v.at[j]], rows_v.at[b], gsem.at[b]
        )

    def write_cp(j, b):
        return pltpu.make_async_copy(
            rows_v.at[b], out_hbm.at[pl.ds(base + j * CHUNK, CHUNK)], wsem.at[b]
        )

    gather_cp(0, 0).start()
    gather_cp(1, 1).start()

    def body(g, carry):
        j0 = g * 2
        gather_cp(j0, 0).wait()
        write_cp(j0, 0).start()
        gather_cp(j0 + 1, 1).wait()
        write_cp(j0 + 1, 1).start()
        write_cp(j0, 0).wait()
        gather_cp(j0 + 2, 0).start()
        write_cp(j0 + 1, 1).wait()
        gather_cp(j0 + 3, 1).start()
        return carry

    lax.fori_loop(0, NUM_GROUPS - 1, body, 0)
    j0 = (NUM_GROUPS - 1) * 2
    gather_cp(j0, 0).wait()
    write_cp(j0, 0).start()
    gather_cp(j0 + 1, 1).wait()
    write_cp(j0 + 1, 1).start()
    write_cp(j0, 0).wait()
    write_cp(j0 + 1, 1).wait()


def kernel(x, weight):
    w2 = _transpose_kernel(weight.T)
    table = w2.reshape(VOCAB_PAD, D_MODEL)
    v = x.reshape(NUM_WORKERS, NUM_CHUNKS, CHUNK).astype(jnp.int32)
    q = v & (VBLK - 1)
    r = (v - q) + 2 * (q & (VBLK // 2 - 1)) + (q >> (VBLK // 2).bit_length() - 1)
    out = _gather_kernel(r, table)
    return out.reshape(x.shape + (D_MODEL,))


# speedup vs baseline: 1.0096x; 1.0085x over previous
import functools

import jax
import jax.numpy as jnp
from jax import lax
from jax.experimental import pallas as pl
from jax.experimental.pallas import tpu as pltpu
from jax.experimental.pallas import tpu_sc as plsc

VOCAB = 1000000
D_MODEL = 64
B_TOTAL = 16384 * 50
NUM_WORKERS = 32
B_PER_W = B_TOTAL // NUM_WORKERS
CHUNK = 512
NUM_CHUNKS = B_PER_W // CHUNK
NUM_GROUPS = NUM_CHUNKS // 2

VBLK = 32768
N_VBLK = -(-VOCAB // VBLK)
VOCAB_PAD = N_VBLK * VBLK

_mesh = plsc.VectorSubcoreMesh(core_axis_name="c", subcore_axis_name="s")


def _transpose_body(wt_ref, out_ref):
    a = wt_ref[:, : VBLK // 2]
    b = wt_ref[:, VBLK // 2 :]
    out_ref[:, :D_MODEL] = jnp.transpose(a)
    out_ref[:, D_MODEL:] = jnp.transpose(b)


_transpose_kernel = pl.pallas_call(
    _transpose_body,
    grid=(N_VBLK,),
    in_specs=[pl.BlockSpec((D_MODEL, VBLK), lambda i: (0, i))],
    out_specs=pl.BlockSpec((VBLK // 2, 2 * D_MODEL), lambda i: (i, 0)),
    out_shape=jax.ShapeDtypeStruct((VOCAB_PAD // 2, 2 * D_MODEL), jnp.float32),
    compiler_params=pltpu.CompilerParams(
        dimension_semantics=("arbitrary",),
    ),
)


@functools.partial(
    pl.kernel,
    mesh=_mesh,
    out_type=jax.ShapeDtypeStruct((B_TOTAL, D_MODEL), jnp.float32),
    scratch_types=[
        pltpu.VMEM((NUM_CHUNKS, CHUNK), jnp.int32),
        pltpu.VMEM((2, CHUNK, D_MODEL), jnp.float32),
        pltpu.SemaphoreType.DMA((2,)),
    ],
    compiler_params=pltpu.CompilerParams(use_tc_tiling_on_sc=False),
)
def _gather_kernel(idx_hbm, table_hbm, out_hbm, idx_v, rows_v, gsem):
    wid = lax.axis_index("s") * 2 + lax.axis_index("c")
    base = wid * B_PER_W
    pltpu.sync_copy(idx_hbm.at[wid], idx_v)

    def fire(j, b):
        pltpu.make_async_copy(
            table_hbm.at[idx_v.at[j]], rows_v.at[b], gsem.at[b]
        ).start()

    def drain_and_write(j, b):
        pltpu.make_async_copy(
            table_hbm.at[idx_v.at[j]], rows_v.at[b], gsem.at[b]
        ).wait()
        pltpu.sync_copy(rows_v.at[b], out_hbm.at[pl.ds(base + j * CHUNK, CHUNK)])

    fire(0, 0)

    def body(g, carry):
        j0 = g * 2
        fire(j0 + 1, 1)
        drain_and_write(j0, 0)
        fire(j0 + 2, 0)
        drain_and_write(j0 + 1, 1)
        return carry

    lax.fori_loop(0, NUM_GROUPS - 1, body, 0)
    j0 = (NUM_GROUPS - 1) * 2
    fire(j0 + 1, 1)
    drain_and_write(j0, 0)
    drain_and_write(j0 + 1, 1)


def kernel(x, weight):
    w2 = _transpose_kernel(weight.T)
    table = w2.reshape(VOCAB_PAD, D_MODEL)
    v = x.reshape(NUM_WORKERS, NUM_CHUNKS, CHUNK).astype(jnp.int32)
    q = v & (VBLK - 1)
    r = (v - q) + 2 * (q & (VBLK // 2 - 1)) + (q >> (VBLK // 2).bit_length() - 1)
    out = _gather_kernel(r, table)
    return out.reshape(x.shape + (D_MODEL,))
